# Initial kernel scaffold; baseline (speedup 1.0000x reference)
import jax
import jax.numpy as jnp
from jax.experimental import pallas as pl


def kernel(h, edge_index, params):
    raise NotImplementedError("write your pallas kernel here")


# speedup vs baseline: 36.5938x; 36.5938x over previous
import functools

import jax
import jax.numpy as jnp
import numpy as np
from jax import lax
from jax.experimental import pallas as pl
from jax.experimental.pallas import tpu as pltpu
from jax.experimental.pallas import tpu_sc as plsc

_N = 10000
_E = 320000
_D = 128
_H = 8
_DH = 16

_NC = 2
_NS = 16
_NW = _NC * _NS
_EPW = _E // _NW
_BC = 16
_NCHUNK = _EPW // _BC
_NP = 10240
_RPT = _NP // _NS
_ZB = 16


_BN = 1000


def _qkv_body(x_ref, wq_ref, wk_ref, wv_ref, bq_ref, bk_ref, bv_ref,
              q_ref, k_ref, v_ref):
    x = x_ref[...]
    q_ref[...] = jnp.dot(x, wq_ref[...], preferred_element_type=jnp.float32) + bq_ref[...]
    k_ref[...] = (jnp.dot(x, wk_ref[...], preferred_element_type=jnp.float32)
                  + bk_ref[...]) * 0.25
    v_ref[...] = jnp.dot(x, wv_ref[...], preferred_element_type=jnp.float32) + bv_ref[...]


def _qkv(x, wq, wk, wv, bq, bk, bv):
    grid = (_N // _BN,)
    row_spec = pl.BlockSpec((_BN, _D), lambda i: (i, 0))
    full_spec = pl.BlockSpec((_D, _D), lambda i: (0, 0))
    bias_spec = pl.BlockSpec((1, _D), lambda i: (0, 0))
    out = jax.ShapeDtypeStruct((_N, _D), jnp.float32)
    return pl.pallas_call(
        _qkv_body,
        grid=grid,
        in_specs=[row_spec, full_spec, full_spec, full_spec,
                  bias_spec, bias_spec, bias_spec],
        out_specs=[row_spec, row_spec, row_spec],
        out_shape=[out, out, out],
    )(x, wq, wk, wv, bq.reshape(1, _D), bk.reshape(1, _D), bv.reshape(1, _D))


_GDN = lax.GatherDimensionNumbers(offset_dims=(), collapsed_slice_dims=(0,),
                                  start_index_map=(0,))


def _vshuffle(x, idx):
    return lax.gather(x, idx[:, None], _GDN, slice_sizes=(1,),
                      mode=lax.GatherScatterMode.PROMISE_IN_BOUNDS)


def _sc_attn_body(q_hbm, k_hbm, v_hbm, src_hbm, dst_hbm,
                  wv_out, z_out,
                  srcv0, dstv0, srcv1, dstv1,
                  kb0, qb0, vb0, kb1, qb1, vb1, wvb, zbuf,
                  acc_wv, acc_z, rsem0, rsem1):
    cid = lax.axis_index("c")
    sid = lax.axis_index("s")
    zero = jnp.zeros((16,), jnp.float32)
    lane = lax.broadcasted_iota(jnp.int32, (16,), 0)

    srcv = (srcv0, srcv1)
    dstv = (dstv0, dstv1)
    kb = (kb0, kb1)
    qb = (qb0, qb1)
    vb = (vb0, vb1)
    rsem = (rsem0, rsem1)

    def zfill(i, _):
        for j in range(8):
            wvb[i, pl.ds(16 * j, 16)] = zero
        zbuf[i, :] = zero
        return 0
    lax.fori_loop(0, _BC, zfill, 0)

    r0 = sid * _RPT

    def zcopy(j, _):
        pltpu.sync_copy(wvb, acc_wv.at[pl.ds(r0 + j * _BC, _BC)])
        pltpu.sync_copy(zbuf, acc_z.at[pl.ds(r0 + j * _BC, _BC)])
        return 0
    lax.fori_loop(0, _RPT // _BC, zcopy, 0)
    plsc.subcore_barrier()

    base = (cid * _NS + sid) * _EPW
    bfly = [lane ^ s for s in (8, 4, 2, 1)]

    def issue_rows(b):
        return (pltpu.async_copy(k_hbm.at[srcv[b]], kb[b], rsem[b]),
                pltpu.async_copy(q_hbm.at[dstv[b]], qb[b], rsem[b]),
                pltpu.async_copy(v_hbm.at[srcv[b]], vb[b], rsem[b]))

    def load_idx(g, b):
        off = base + g * _BC
        pltpu.sync_copy(src_hbm.at[pl.ds(off, _BC)], srcv[b])
        pltpu.sync_copy(dst_hbm.at[pl.ds(off, _BC)], dstv[b])

    def compute_scatter(b):
        def edge(e, _):
            zrow = zero
            for hh in range(_H):
                kh = kb[b][e, pl.ds(16 * hh, 16)]
                qh = qb[b][e, pl.ds(16 * hh, 16)]
                sv = kh * qh
                for ix in bfly:
                    sv = sv + _vshuffle(sv, ix)
                sv = jnp.minimum(jnp.maximum(sv, -5.0), 5.0)
                ev = jnp.exp(sv)
                wvb[e, pl.ds(16 * hh, 16)] = ev * vb[b][e, pl.ds(16 * hh, 16)]
                zrow = jnp.where(lane == hh, ev, zrow)
            zbuf[e, :] = zrow
            return 0

        lax.fori_loop(0, _BC, edge, 0)
        pltpu.sync_copy(wvb, acc_wv.at[dstv[b]], add=True)
        pltpu.sync_copy(zbuf, acc_z.at[dstv[b]], add=True)

    load_idx(0, 0)
    for h in issue_rows(0):
        h.wait()

    def pair(go, _):
        for b in range(2):
            g = 2 * go + b
            b1 = 1 - b
            load_idx(g + 1, b1)
            hs = issue_rows(b1)
            compute_scatter(b)
            for h in hs:
                h.wait()
        return 0

    lax.fori_loop(0, (_NCHUNK - 1) // 2, pair, 0)

    compute_scatter(0)
    plsc.subcore_barrier()

    pltpu.sync_copy(acc_wv.at[pl.ds(r0, _RPT)],
                    wv_out.at[cid, pl.ds(r0, _RPT)])
    pltpu.sync_copy(acc_z.at[pl.ds(r0, _RPT)],
                    z_out.at[cid, pl.ds(r0, _RPT)])


@jax.jit
def _sc_attn(q, k, v, src, dst):
    mesh = plsc.VectorSubcoreMesh(core_axis_name="c", subcore_axis_name="s")
    fn = pl.kernel(
        _sc_attn_body,
        out_type=[jax.ShapeDtypeStruct((_NC, _NP, _D), jnp.float32),
                  jax.ShapeDtypeStruct((_NC, _NP, _DH), jnp.float32)],
        mesh=mesh,
        scratch_types=[
            pltpu.VMEM((_BC,), jnp.int32),
            pltpu.VMEM((_BC,), jnp.int32),
            pltpu.VMEM((_BC,), jnp.int32),
            pltpu.VMEM((_BC,), jnp.int32),
            pltpu.VMEM((_BC, _D), jnp.float32),
            pltpu.VMEM((_BC, _D), jnp.float32),
            pltpu.VMEM((_BC, _D), jnp.float32),
            pltpu.VMEM((_BC, _D), jnp.float32),
            pltpu.VMEM((_BC, _D), jnp.float32),
            pltpu.VMEM((_BC, _D), jnp.float32),
            pltpu.VMEM((_BC, _D), jnp.float32),
            pltpu.VMEM((_BC, _DH), jnp.float32),
            pltpu.VMEM_SHARED((_NP, _D), jnp.float32),
            pltpu.VMEM_SHARED((_NP, _DH), jnp.float32),
            pltpu.SemaphoreType.DMA,
            pltpu.SemaphoreType.DMA,
        ],
        compiler_params=pltpu.CompilerParams(use_tc_tiling_on_sc=False),
    )
    return fn(q, k, v, src, dst)


def _post_body(x_ref, wv_ref, z_ref, r_ref, wo_ref, bo_ref, g1_ref, b1_ref,
               w1_ref, bf1_ref, w2_ref, bf2_ref, g2_ref, b2_ref, o_ref):
    wv = wv_ref[0] + wv_ref[1]
    z = z_ref[0] + z_ref[1]
    zexp = jnp.dot(z, r_ref[...], preferred_element_type=jnp.float32)
    h_att = wv / (zexp + 1e-6)
    h_o = jnp.dot(h_att, wo_ref[...], preferred_element_type=jnp.float32) + bo_ref[...]
    h1 = x_ref[...] + h_o
    mu = jnp.mean(h1, axis=-1, keepdims=True)
    var = jnp.mean((h1 - mu) * (h1 - mu), axis=-1, keepdims=True)
    h1 = (h1 - mu) * lax.rsqrt(var + 1e-5) * g1_ref[...] + b1_ref[...]
    hf = jnp.dot(h1, w1_ref[...], preferred_element_type=jnp.float32) + bf1_ref[...]
    hf = jnp.maximum(hf, 0.0)
    hf = jnp.dot(hf, w2_ref[...], preferred_element_type=jnp.float32) + bf2_ref[...]
    h2 = h1 + hf
    mu2 = jnp.mean(h2, axis=-1, keepdims=True)
    var2 = jnp.mean((h2 - mu2) * (h2 - mu2), axis=-1, keepdims=True)
    o_ref[...] = (h2 - mu2) * lax.rsqrt(var2 + 1e-5) * g2_ref[...] + b2_ref[...]


_R_EXPAND = np.zeros((_DH, _D), np.float32)
for _hh in range(_H):
    _R_EXPAND[_hh, _hh * _DH:(_hh + 1) * _DH] = 1.0


def _post(x, wv_parts, z_parts, p):
    grid = (_N // _BN,)
    row_spec = pl.BlockSpec((_BN, _D), lambda i: (i, 0))
    wv_spec = pl.BlockSpec((_NC, _BN, _D), lambda i: (0, i, 0))
    z_spec = pl.BlockSpec((_NC, _BN, _DH), lambda i: (0, i, 0))
    r_spec = pl.BlockSpec((_DH, _D), lambda i: (0, 0))
    d_spec = pl.BlockSpec((_D, _D), lambda i: (0, 0))
    b_spec = pl.BlockSpec((1, _D), lambda i: (0, 0))
    w1_spec = pl.BlockSpec((_D, 2 * _D), lambda i: (0, 0))
    b1_spec = pl.BlockSpec((1, 2 * _D), lambda i: (0, 0))
    w2_spec = pl.BlockSpec((2 * _D, _D), lambda i: (0, 0))
    return pl.pallas_call(
        _post_body,
        grid=grid,
        in_specs=[row_spec, wv_spec, z_spec, r_spec, d_spec, b_spec,
                  b_spec, b_spec, w1_spec, b1_spec, w2_spec, b_spec,
                  b_spec, b_spec],
        out_specs=row_spec,
        out_shape=jax.ShapeDtypeStruct((_N, _D), jnp.float32),
    )(x, wv_parts, z_parts, jnp.asarray(_R_EXPAND),
      p['Wo'], p['bo'].reshape(1, _D), p['g1'].reshape(1, _D),
      p['b1'].reshape(1, _D), p['W1'], p['bf1'].reshape(1, 2 * _D),
      p['W2'], p['bf2'].reshape(1, _D), p['g2'].reshape(1, _D),
      p['b2'].reshape(1, _D))


def kernel(h, edge_index, params):
    src = edge_index[0].astype(jnp.int32)
    dst = edge_index[1].astype(jnp.int32)
    x = h
    for p in params:
        q, k, v = _qkv(x, p['Wq'], p['Wk'], p['Wv'], p['bq'], p['bk'], p['bv'])
        wv_parts, z_parts = _sc_attn(q, k, v, src, dst)
        x = _post(x, wv_parts, z_parts, p)
    return x


# speedup vs baseline: 36.5997x; 1.0002x over previous
import functools

import jax
import jax.numpy as jnp
import numpy as np
from jax import lax
from jax.experimental import pallas as pl
from jax.experimental.pallas import tpu as pltpu
from jax.experimental.pallas import tpu_sc as plsc

_N = 10000
_E = 320000
_D = 128
_H = 8
_DH = 16

_NC = 2
_NS = 16
_NW = _NC * _NS
_EPW = _E // _NW
_BC = 16
_NCHUNK = _EPW // _BC
_NP = 10240
_RPT = _NP // _NS
_ZB = 16


_BN = 1000


def _qkv_body(x_ref, wq_ref, wk_ref, wv_ref, bq_ref, bk_ref, bv_ref,
              q_ref, k_ref, v_ref):
    x = x_ref[...]
    q_ref[...] = jnp.dot(x, wq_ref[...], preferred_element_type=jnp.float32) + bq_ref[...]
    k_ref[...] = (jnp.dot(x, wk_ref[...], preferred_element_type=jnp.float32)
                  + bk_ref[...]) * 0.25
    v_ref[...] = jnp.dot(x, wv_ref[...], preferred_element_type=jnp.float32) + bv_ref[...]


def _qkv(x, wq, wk, wv, bq, bk, bv):
    grid = (_N // _BN,)
    row_spec = pl.BlockSpec((_BN, _D), lambda i: (i, 0))
    full_spec = pl.BlockSpec((_D, _D), lambda i: (0, 0))
    bias_spec = pl.BlockSpec((1, _D), lambda i: (0, 0))
    out = jax.ShapeDtypeStruct((_N, _D), jnp.float32)
    return pl.pallas_call(
        _qkv_body,
        grid=grid,
        in_specs=[row_spec, full_spec, full_spec, full_spec,
                  bias_spec, bias_spec, bias_spec],
        out_specs=[row_spec, row_spec, row_spec],
        out_shape=[out, out, out],
    )(x, wq, wk, wv, bq.reshape(1, _D), bk.reshape(1, _D), bv.reshape(1, _D))


_GDN = lax.GatherDimensionNumbers(offset_dims=(), collapsed_slice_dims=(0,),
                                  start_index_map=(0,))


def _vshuffle(x, idx):
    return lax.gather(x, idx[:, None], _GDN, slice_sizes=(1,),
                      mode=lax.GatherScatterMode.PROMISE_IN_BOUNDS)


def _sc_attn_body(q_hbm, k_hbm, v_hbm, src_hbm, dst_hbm,
                  wv_out, z_out,
                  srcv0, dstv0, srcv1, dstv1,
                  kb0, qb0, vb0, kb1, qb1, vb1, wvb, zbuf,
                  acc_wv, acc_z, rsem0, rsem1):
    cid = lax.axis_index("c")
    sid = lax.axis_index("s")
    zero = jnp.zeros((16,), jnp.float32)
    lane = lax.broadcasted_iota(jnp.int32, (16,), 0)

    srcv = (srcv0, srcv1)
    dstv = (dstv0, dstv1)
    kb = (kb0, kb1)
    qb = (qb0, qb1)
    vb = (vb0, vb1)
    rsem = (rsem0, rsem1)

    def zfill(i, _):
        for j in range(8):
            wvb[i, pl.ds(16 * j, 16)] = zero
        zbuf[i, :] = zero
        return 0
    lax.fori_loop(0, _BC, zfill, 0)

    r0 = sid * _RPT

    def zcopy(j, _):
        pltpu.sync_copy(wvb, acc_wv.at[pl.ds(r0 + j * _BC, _BC)])
        pltpu.sync_copy(zbuf, acc_z.at[pl.ds(r0 + j * _BC, _BC)])
        return 0
    lax.fori_loop(0, _RPT // _BC, zcopy, 0)
    plsc.subcore_barrier()

    base = (cid * _NS + sid) * _EPW
    lane15 = jnp.full((16,), 15, jnp.int32)

    def issue_rows(b):
        return (pltpu.async_copy(k_hbm.at[srcv[b]], kb[b], rsem[b]),
                pltpu.async_copy(q_hbm.at[dstv[b]], qb[b], rsem[b]),
                pltpu.async_copy(v_hbm.at[srcv[b]], vb[b], rsem[b]))

    def load_idx(g, b):
        off = base + g * _BC
        pltpu.sync_copy(src_hbm.at[pl.ds(off, _BC)], srcv[b])
        pltpu.sync_copy(dst_hbm.at[pl.ds(off, _BC)], dstv[b])

    def compute_scatter(b):
        def edge(e, _):
            zrow = zero
            for hh in range(_H):
                kh = kb[b][e, pl.ds(16 * hh, 16)]
                qh = qb[b][e, pl.ds(16 * hh, 16)]
                sv = kh * qh
                for step in (1, 2, 4, 8):
                    sv = sv + _vshuffle(sv, jnp.bitwise_xor(lane, step))
                sv = jnp.minimum(jnp.maximum(sv, -5.0), 5.0)
                ev = jnp.exp(sv)
                wvb[e, pl.ds(16 * hh, 16)] = ev * vb[b][e, pl.ds(16 * hh, 16)]
                zrow = jnp.where(lane == hh, ev, zrow)
            zbuf[e, :] = zrow
            return 0

        lax.fori_loop(0, _BC, edge, 0)
        pltpu.sync_copy(wvb, acc_wv.at[dstv[b]], add=True)
        pltpu.sync_copy(zbuf, acc_z.at[dstv[b]], add=True)

    load_idx(0, 0)
    for h in issue_rows(0):
        h.wait()

    def pair(go, _):
        for b in range(2):
            g = 2 * go + b
            b1 = 1 - b
            load_idx(g + 1, b1)
            hs = issue_rows(b1)
            compute_scatter(b)
            for h in hs:
                h.wait()
        return 0

    lax.fori_loop(0, (_NCHUNK - 1) // 2, pair, 0)

    compute_scatter(0)
    plsc.subcore_barrier()

    pltpu.sync_copy(acc_wv.at[pl.ds(r0, _RPT)],
                    wv_out.at[cid, pl.ds(r0, _RPT)])
    pltpu.sync_copy(acc_z.at[pl.ds(r0, _RPT)],
                    z_out.at[cid, pl.ds(r0, _RPT)])


@jax.jit
def _sc_attn(q, k, v, src, dst):
    mesh = plsc.VectorSubcoreMesh(core_axis_name="c", subcore_axis_name="s")
    fn = pl.kernel(
        _sc_attn_body,
        out_type=[jax.ShapeDtypeStruct((_NC, _NP, _D), jnp.float32),
                  jax.ShapeDtypeStruct((_NC, _NP, _DH), jnp.float32)],
        mesh=mesh,
        scratch_types=[
            pltpu.VMEM((_BC,), jnp.int32),
            pltpu.VMEM((_BC,), jnp.int32),
            pltpu.VMEM((_BC,), jnp.int32),
            pltpu.VMEM((_BC,), jnp.int32),
            pltpu.VMEM((_BC, _D), jnp.float32),
            pltpu.VMEM((_BC, _D), jnp.float32),
            pltpu.VMEM((_BC, _D), jnp.float32),
            pltpu.VMEM((_BC, _D), jnp.float32),
            pltpu.VMEM((_BC, _D), jnp.float32),
            pltpu.VMEM((_BC, _D), jnp.float32),
            pltpu.VMEM((_BC, _D), jnp.float32),
            pltpu.VMEM((_BC, _DH), jnp.float32),
            pltpu.VMEM_SHARED((_NP, _D), jnp.float32),
            pltpu.VMEM_SHARED((_NP, _DH), jnp.float32),
            pltpu.SemaphoreType.DMA,
            pltpu.SemaphoreType.DMA,
        ],
        compiler_params=pltpu.CompilerParams(use_tc_tiling_on_sc=False),
    )
    return fn(q, k, v, src, dst)


def _post_body(x_ref, wv_ref, z_ref, r_ref, wo_ref, bo_ref, g1_ref, b1_ref,
               w1_ref, bf1_ref, w2_ref, bf2_ref, g2_ref, b2_ref, o_ref):
    wv = wv_ref[0] + wv_ref[1]
    z = z_ref[0] + z_ref[1]
    zexp = jnp.dot(z, r_ref[...], preferred_element_type=jnp.float32)
    h_att = wv / (zexp + 1e-6)
    h_o = jnp.dot(h_att, wo_ref[...], preferred_element_type=jnp.float32) + bo_ref[...]
    h1 = x_ref[...] + h_o
    mu = jnp.mean(h1, axis=-1, keepdims=True)
    var = jnp.mean((h1 - mu) * (h1 - mu), axis=-1, keepdims=True)
    h1 = (h1 - mu) * lax.rsqrt(var + 1e-5) * g1_ref[...] + b1_ref[...]
    hf = jnp.dot(h1, w1_ref[...], preferred_element_type=jnp.float32) + bf1_ref[...]
    hf = jnp.maximum(hf, 0.0)
    hf = jnp.dot(hf, w2_ref[...], preferred_element_type=jnp.float32) + bf2_ref[...]
    h2 = h1 + hf
    mu2 = jnp.mean(h2, axis=-1, keepdims=True)
    var2 = jnp.mean((h2 - mu2) * (h2 - mu2), axis=-1, keepdims=True)
    o_ref[...] = (h2 - mu2) * lax.rsqrt(var2 + 1e-5) * g2_ref[...] + b2_ref[...]


_R_EXPAND = np.zeros((_DH, _D), np.float32)
for _hh in range(_H):
    _R_EXPAND[_hh, _hh * _DH:(_hh + 1) * _DH] = 1.0


def _post(x, wv_parts, z_parts, p):
    grid = (_N // _BN,)
    row_spec = pl.BlockSpec((_BN, _D), lambda i: (i, 0))
    wv_spec = pl.BlockSpec((_NC, _BN, _D), lambda i: (0, i, 0))
    z_spec = pl.BlockSpec((_NC, _BN, _DH), lambda i: (0, i, 0))
    r_spec = pl.BlockSpec((_DH, _D), lambda i: (0, 0))
    d_spec = pl.BlockSpec((_D, _D), lambda i: (0, 0))
    b_spec = pl.BlockSpec((1, _D), lambda i: (0, 0))
    w1_spec = pl.BlockSpec((_D, 2 * _D), lambda i: (0, 0))
    b1_spec = pl.BlockSpec((1, 2 * _D), lambda i: (0, 0))
    w2_spec = pl.BlockSpec((2 * _D, _D), lambda i: (0, 0))
    return pl.pallas_call(
        _post_body,
        grid=grid,
        in_specs=[row_spec, wv_spec, z_spec, r_spec, d_spec, b_spec,
                  b_spec, b_spec, w1_spec, b1_spec, w2_spec, b_spec,
                  b_spec, b_spec],
        out_specs=row_spec,
        out_shape=jax.ShapeDtypeStruct((_N, _D), jnp.float32),
    )(x, wv_parts, z_parts, jnp.asarray(_R_EXPAND),
      p['Wo'], p['bo'].reshape(1, _D), p['g1'].reshape(1, _D),
      p['b1'].reshape(1, _D), p['W1'], p['bf1'].reshape(1, 2 * _D),
      p['W2'], p['bf2'].reshape(1, _D), p['g2'].reshape(1, _D),
      p['b2'].reshape(1, _D))


def kernel(h, edge_index, params):
    src = edge_index[0].astype(jnp.int32)
    dst = edge_index[1].astype(jnp.int32)
    x = h
    for p in params:
        q, k, v = _qkv(x, p['Wq'], p['Wk'], p['Wv'], p['bq'], p['bk'], p['bv'])
        wv_parts, z_parts = _sc_attn(q, k, v, src, dst)
        x = _post(x, wv_parts, z_parts, p)
    return x


# speedup vs baseline: 56.3821x; 1.5405x over previous
import functools

import jax
import jax.numpy as jnp
import numpy as np
from jax import lax
from jax.experimental import pallas as pl
from jax.experimental.pallas import tpu as pltpu
from jax.experimental.pallas import tpu_sc as plsc

_N = 10000
_E = 320000
_D = 128
_H = 8
_DH = 16

_NC = 2
_NS = 16
_NW = _NC * _NS
_EPW = _E // _NW
_BC = 40
_NCHUNK = _EPW // _BC
_NP = 10240
_RPT = _NP // _NS
_ZB = 16


_BN = 1000


def _qkv_body(x_ref, wq_ref, wk_ref, wv_ref, bq_ref, bk_ref, bv_ref,
              q_ref, k_ref, v_ref):
    x = x_ref[...]
    q_ref[...] = jnp.dot(x, wq_ref[...], preferred_element_type=jnp.float32) + bq_ref[...]
    k_ref[...] = (jnp.dot(x, wk_ref[...], preferred_element_type=jnp.float32)
                  + bk_ref[...]) * 0.25
    v_ref[...] = jnp.dot(x, wv_ref[...], preferred_element_type=jnp.float32) + bv_ref[...]


def _qkv(x, wq, wk, wv, bq, bk, bv):
    grid = (_N // _BN,)
    row_spec = pl.BlockSpec((_BN, _D), lambda i: (i, 0))
    full_spec = pl.BlockSpec((_D, _D), lambda i: (0, 0))
    bias_spec = pl.BlockSpec((1, _D), lambda i: (0, 0))
    out = jax.ShapeDtypeStruct((_N, _D), jnp.float32)
    return pl.pallas_call(
        _qkv_body,
        grid=grid,
        in_specs=[row_spec, full_spec, full_spec, full_spec,
                  bias_spec, bias_spec, bias_spec],
        out_specs=[row_spec, row_spec, row_spec],
        out_shape=[out, out, out],
    )(x, wq, wk, wv, bq.reshape(1, _D), bk.reshape(1, _D), bv.reshape(1, _D))


_GDN = lax.GatherDimensionNumbers(offset_dims=(), collapsed_slice_dims=(0,),
                                  start_index_map=(0,))


def _vshuffle(x, idx):
    return lax.gather(x, idx[:, None], _GDN, slice_sizes=(1,),
                      mode=lax.GatherScatterMode.PROMISE_IN_BOUNDS)


def _sc_attn_body(q_hbm, k_hbm, v_hbm, src_hbm, dst_hbm,
                  wv_out, z_out,
                  srcv0, dstv0, srcv1, dstv1,
                  kb0, qb0, vb0, kb1, qb1, vb1, wvb, zbuf,
                  acc_wv, acc_z, rsem0, rsem1):
    cid = lax.axis_index("c")
    sid = lax.axis_index("s")
    zero = jnp.zeros((16,), jnp.float32)
    lane = lax.broadcasted_iota(jnp.int32, (16,), 0)

    srcv = (srcv0, srcv1)
    dstv = (dstv0, dstv1)
    kb = (kb0, kb1)
    qb = (qb0, qb1)
    vb = (vb0, vb1)
    rsem = (rsem0, rsem1)

    def zfill(i, _):
        for j in range(8):
            wvb[i, pl.ds(16 * j, 16)] = zero
        zbuf[i, :] = zero
        return 0
    lax.fori_loop(0, _BC, zfill, 0)

    r0 = sid * _RPT

    def zcopy(j, _):
        pltpu.sync_copy(wvb, acc_wv.at[pl.ds(r0 + j * _BC, _BC)])
        pltpu.sync_copy(zbuf, acc_z.at[pl.ds(r0 + j * _BC, _BC)])
        return 0
    lax.fori_loop(0, _RPT // _BC, zcopy, 0)
    plsc.subcore_barrier()

    base = (cid * _NS + sid) * _EPW
    lane15 = jnp.full((16,), 15, jnp.int32)

    def issue_rows(b):
        return (pltpu.async_copy(k_hbm.at[srcv[b]], kb[b], rsem[b]),
                pltpu.async_copy(q_hbm.at[dstv[b]], qb[b], rsem[b]),
                pltpu.async_copy(v_hbm.at[srcv[b]], vb[b], rsem[b]))

    def load_idx(g, b):
        off = base + g * _BC
        pltpu.sync_copy(src_hbm.at[pl.ds(off, _BC)], srcv[b])
        pltpu.sync_copy(dst_hbm.at[pl.ds(off, _BC)], dstv[b])

    def compute_scatter(b):
        def edge(e, _):
            zrow = zero
            for hh in range(_H):
                kh = kb[b][e, pl.ds(16 * hh, 16)]
                qh = qb[b][e, pl.ds(16 * hh, 16)]
                sv = kh * qh
                for step in (1, 2, 4, 8):
                    sv = sv + _vshuffle(sv, jnp.bitwise_xor(lane, step))
                sv = jnp.minimum(jnp.maximum(sv, -5.0), 5.0)
                ev = jnp.exp(sv)
                wvb[e, pl.ds(16 * hh, 16)] = ev * vb[b][e, pl.ds(16 * hh, 16)]
                zrow = jnp.where(lane == hh, ev, zrow)
            zbuf[e, :] = zrow
            return 0

        lax.fori_loop(0, _BC, edge, 0)
        pltpu.sync_copy(wvb, acc_wv.at[dstv[b]], add=True)
        pltpu.sync_copy(zbuf, acc_z.at[dstv[b]], add=True)

    load_idx(0, 0)
    for h in issue_rows(0):
        h.wait()

    def pair(go, _):
        for b in range(2):
            g = 2 * go + b
            b1 = 1 - b
            load_idx(g + 1, b1)
            hs = issue_rows(b1)
            compute_scatter(b)
            for h in hs:
                h.wait()
        return 0

    lax.fori_loop(0, (_NCHUNK - 2) // 2, pair, 0)

    load_idx(_NCHUNK - 1, 1)
    hs = issue_rows(1)
    compute_scatter(0)
    for h in hs:
        h.wait()
    compute_scatter(1)
    plsc.subcore_barrier()

    pltpu.sync_copy(acc_wv.at[pl.ds(r0, _RPT)],
                    wv_out.at[cid, pl.ds(r0, _RPT)])
    pltpu.sync_copy(acc_z.at[pl.ds(r0, _RPT)],
                    z_out.at[cid, pl.ds(r0, _RPT)])


@jax.jit
def _sc_attn(q, k, v, src, dst):
    mesh = plsc.VectorSubcoreMesh(core_axis_name="c", subcore_axis_name="s")
    fn = pl.kernel(
        _sc_attn_body,
        out_type=[jax.ShapeDtypeStruct((_NC, _NP, _D), jnp.float32),
                  jax.ShapeDtypeStruct((_NC, _NP, _DH), jnp.float32)],
        mesh=mesh,
        scratch_types=[
            pltpu.VMEM((_BC,), jnp.int32),
            pltpu.VMEM((_BC,), jnp.int32),
            pltpu.VMEM((_BC,), jnp.int32),
            pltpu.VMEM((_BC,), jnp.int32),
            pltpu.VMEM((_BC, _D), jnp.float32),
            pltpu.VMEM((_BC, _D), jnp.float32),
            pltpu.VMEM((_BC, _D), jnp.float32),
            pltpu.VMEM((_BC, _D), jnp.float32),
            pltpu.VMEM((_BC, _D), jnp.float32),
            pltpu.VMEM((_BC, _D), jnp.float32),
            pltpu.VMEM((_BC, _D), jnp.float32),
            pltpu.VMEM((_BC, _DH), jnp.float32),
            pltpu.VMEM_SHARED((_NP, _D), jnp.float32),
            pltpu.VMEM_SHARED((_NP, _DH), jnp.float32),
            pltpu.SemaphoreType.DMA,
            pltpu.SemaphoreType.DMA,
        ],
        compiler_params=pltpu.CompilerParams(use_tc_tiling_on_sc=False),
    )
    return fn(q, k, v, src, dst)


def _post_body(x_ref, wv_ref, z_ref, r_ref, wo_ref, bo_ref, g1_ref, b1_ref,
               w1_ref, bf1_ref, w2_ref, bf2_ref, g2_ref, b2_ref, o_ref):
    wv = wv_ref[0] + wv_ref[1]
    z = z_ref[0] + z_ref[1]
    zexp = jnp.dot(z, r_ref[...], preferred_element_type=jnp.float32)
    h_att = wv / (zexp + 1e-6)
    h_o = jnp.dot(h_att, wo_ref[...], preferred_element_type=jnp.float32) + bo_ref[...]
    h1 = x_ref[...] + h_o
    mu = jnp.mean(h1, axis=-1, keepdims=True)
    var = jnp.mean((h1 - mu) * (h1 - mu), axis=-1, keepdims=True)
    h1 = (h1 - mu) * lax.rsqrt(var + 1e-5) * g1_ref[...] + b1_ref[...]
    hf = jnp.dot(h1, w1_ref[...], preferred_element_type=jnp.float32) + bf1_ref[...]
    hf = jnp.maximum(hf, 0.0)
    hf = jnp.dot(hf, w2_ref[...], preferred_element_type=jnp.float32) + bf2_ref[...]
    h2 = h1 + hf
    mu2 = jnp.mean(h2, axis=-1, keepdims=True)
    var2 = jnp.mean((h2 - mu2) * (h2 - mu2), axis=-1, keepdims=True)
    o_ref[...] = (h2 - mu2) * lax.rsqrt(var2 + 1e-5) * g2_ref[...] + b2_ref[...]


_R_EXPAND = np.zeros((_DH, _D), np.float32)
for _hh in range(_H):
    _R_EXPAND[_hh, _hh * _DH:(_hh + 1) * _DH] = 1.0


def _post(x, wv_parts, z_parts, p):
    grid = (_N // _BN,)
    row_spec = pl.BlockSpec((_BN, _D), lambda i: (i, 0))
    wv_spec = pl.BlockSpec((_NC, _BN, _D), lambda i: (0, i, 0))
    z_spec = pl.BlockSpec((_NC, _BN, _DH), lambda i: (0, i, 0))
    r_spec = pl.BlockSpec((_DH, _D), lambda i: (0, 0))
    d_spec = pl.BlockSpec((_D, _D), lambda i: (0, 0))
    b_spec = pl.BlockSpec((1, _D), lambda i: (0, 0))
    w1_spec = pl.BlockSpec((_D, 2 * _D), lambda i: (0, 0))
    b1_spec = pl.BlockSpec((1, 2 * _D), lambda i: (0, 0))
    w2_spec = pl.BlockSpec((2 * _D, _D), lambda i: (0, 0))
    return pl.pallas_call(
        _post_body,
        grid=grid,
        in_specs=[row_spec, wv_spec, z_spec, r_spec, d_spec, b_spec,
                  b_spec, b_spec, w1_spec, b1_spec, w2_spec, b_spec,
                  b_spec, b_spec],
        out_specs=row_spec,
        out_shape=jax.ShapeDtypeStruct((_N, _D), jnp.float32),
    )(x, wv_parts, z_parts, jnp.asarray(_R_EXPAND),
      p['Wo'], p['bo'].reshape(1, _D), p['g1'].reshape(1, _D),
      p['b1'].reshape(1, _D), p['W1'], p['bf1'].reshape(1, 2 * _D),
      p['W2'], p['bf2'].reshape(1, _D), p['g2'].reshape(1, _D),
      p['b2'].reshape(1, _D))


def kernel(h, edge_index, params):
    src = edge_index[0].astype(jnp.int32)
    dst = edge_index[1].astype(jnp.int32)
    x = h
    for p in params:
        q, k, v = _qkv(x, p['Wq'], p['Wk'], p['Wv'], p['bq'], p['bk'], p['bv'])
        wv_parts, z_parts = _sc_attn(q, k, v, src, dst)
        x = _post(x, wv_parts, z_parts, p)
    return x
